# baseline (speedup 1.0000x reference)
import jax
import jax.numpy as jnp
from jax import lax
from jax.experimental import pallas as pl
from jax.experimental.pallas import tpu as pltpu
from jax.experimental.pallas import tpu_sc as plsc

B, D, H, W = 8, 256, 24, 24
N = B * H * W
V = 8192
KT = 512
NK = V // KT
TT = 512
NT = N // TT

_BIG = float(2**30)


def _argmin_body(ht2_ref, cb_ref, fn_ref, cn_ref, out_ref):
    ht2 = ht2_ref[...]
    fn = fn_ref[0]
    w2 = jnp.where(
        lax.broadcasted_iota(jnp.int32, (2, KT), 0) == 0, 1.0,
        lax.broadcasted_iota(jnp.int32, (2, KT), 1).astype(jnp.float32))
    rmin = None
    ridx = None
    for k in range(NK):
        cbk = cb_ref[pl.ds(k * KT, KT), :]
        mm2 = lax.dot_general(cbk, ht2, (((1,), (0,)), ((), ())),
                              preferred_element_type=jnp.float32)
        cn = cn_ref[pl.ds(k * KT, KT), :]
        d2 = (fn + mm2) + cn
        lmin = jnp.min(d2, axis=0, keepdims=True)
        ind = jnp.where(d2 == lmin, 1.0, 0.0)
        ext = lax.dot_general(w2, ind, (((1,), (0,)), ((), ())),
                              precision=lax.Precision.HIGHEST,
                              preferred_element_type=jnp.float32)
        cnt = ext[0:1]
        sidx = ext[1:2]

        def _tie_fallback(d2=d2, lmin=lmin):
            iot = lax.broadcasted_iota(
                jnp.int32, (KT, TT), 0).astype(jnp.float32)
            return jnp.min(jnp.where(d2 == lmin, iot, _BIG),
                           axis=0, keepdims=True)

        lidx = lax.cond(jnp.any(cnt > 1.0),
                        _tie_fallback, lambda: sidx) + float(k * KT)
        if k == 0:
            rmin, ridx = lmin, lidx
        else:
            upd = lmin < rmin
            rmin = jnp.where(upd, lmin, rmin)
            ridx = jnp.where(upd, lidx, ridx)
    out_ref[0] = ridx.astype(jnp.int32)


def _nearest_codes(ht, cb, fn, cn):
    return pl.pallas_call(
        _argmin_body,
        grid=(NT,),
        in_specs=[
            pl.BlockSpec((D, TT), lambda t: (0, t)),
            pl.BlockSpec((V, D), lambda t: (0, 0)),
            pl.BlockSpec((1, 1, TT), lambda t: (t, 0, 0)),
            pl.BlockSpec((V, 1), lambda t: (0, 0)),
        ],
        out_specs=pl.BlockSpec((1, 1, TT), lambda t: (t, 0, 0)),
        out_shape=jax.ShapeDtypeStruct((NT, 1, TT), jnp.int32),
        compiler_params=pltpu.CompilerParams(
            dimension_semantics=("arbitrary",)),
    )(ht, cb, fn, cn)


_NC = 2
_NS = 16
_NW = _NC * _NS
_ROWS_PER_W = N // _NW
_CHUNK = 72
_NCHUNK = _ROWS_PER_W // _CHUNK


def _gather_body(idx_hbm, cb_hbm, out_hbm, idx_v, rows_v, sem):
    wid = lax.axis_index("s") * _NC + lax.axis_index("c")
    base = wid * _ROWS_PER_W
    pltpu.sync_copy(idx_hbm.at[pl.ds(wid * _NCHUNK, _NCHUNK)], idx_v)
    for c in range(_NCHUNK):
        pltpu.async_copy(cb_hbm.at[idx_v.at[c]], rows_v, sem).wait()
        pltpu.sync_copy(rows_v, out_hbm.at[pl.ds(base + c * _CHUNK, _CHUNK)])


def _gather_rows(idx2d, cb):
    return pl.kernel(
        _gather_body,
        mesh=plsc.VectorSubcoreMesh(core_axis_name="c", subcore_axis_name="s"),
        out_type=jax.ShapeDtypeStruct((N, D), jnp.float32),
        scratch_types=[
            pltpu.VMEM((_NCHUNK, _CHUNK), jnp.int32),
            pltpu.VMEM((_CHUNK, D), jnp.float32),
            pltpu.SemaphoreType.DMA,
        ],
    )(idx2d, cb)


def kernel(h, cb):
    flat = jnp.transpose(h, (0, 2, 3, 1)).reshape(N, D)
    ht2 = -2.0 * jnp.transpose(h.reshape(B, D, H * W), (1, 0, 2)).reshape(D, N)
    fn = jnp.sum(flat * flat, axis=1).reshape(NT, 1, TT)
    cn = jnp.sum(cb * cb, axis=1).reshape(V, 1)
    idx = _nearest_codes(ht2, cb, fn, cn)
    idx_flat = idx.reshape(N)
    q = _gather_rows(idx_flat.reshape(_NW * _NCHUNK, _CHUNK), cb)
    z = idx_flat.reshape(B, H, W)
    return (z, q.reshape(B, H, W, D))


# speedup vs baseline: 2.3357x; 2.3357x over previous
import jax
import jax.numpy as jnp
from jax import lax
from jax.experimental import pallas as pl
from jax.experimental.pallas import tpu as pltpu
from jax.experimental.pallas import tpu_sc as plsc

B, D, H, W = 8, 256, 24, 24
N = B * H * W
V = 8192
KT = 512
NK = V // KT
TT = 512
NT = N // TT

_BIG = float(2**30)


def _argmin_body(ht2_ref, cb_ref, fn_ref, cn_ref, out_ref):
    ht2 = ht2_ref[...]
    fn = fn_ref[0]
    rmin = None
    ridx = None
    for k in range(NK):
        cbk = cb_ref[pl.ds(k * KT, KT), :]
        mm2 = lax.dot_general(cbk, ht2, (((1,), (0,)), ((), ())),
                              preferred_element_type=jnp.float32)
        cn = cn_ref[pl.ds(k * KT, KT), :]
        d2 = (fn + mm2) + cn
        lmin = jnp.min(d2, axis=0, keepdims=True)
        iot = lax.broadcasted_iota(jnp.int32, (KT, TT), 0).astype(jnp.float32)
        lidx = jnp.min(jnp.where(d2 == lmin, iot, _BIG),
                       axis=0, keepdims=True) + float(k * KT)
        if k == 0:
            rmin, ridx = lmin, lidx
        else:
            upd = lmin < rmin
            rmin = jnp.where(upd, lmin, rmin)
            ridx = jnp.where(upd, lidx, ridx)
    out_ref[0] = ridx.astype(jnp.int32)


def _nearest_codes(ht, cb, fn, cn):
    return pl.pallas_call(
        _argmin_body,
        grid=(NT,),
        in_specs=[
            pl.BlockSpec((D, TT), lambda t: (0, t)),
            pl.BlockSpec((V, D), lambda t: (0, 0)),
            pl.BlockSpec((1, 1, TT), lambda t: (t, 0, 0)),
            pl.BlockSpec((V, 1), lambda t: (0, 0)),
        ],
        out_specs=pl.BlockSpec((1, 1, TT), lambda t: (t, 0, 0)),
        out_shape=jax.ShapeDtypeStruct((NT, 1, TT), jnp.int32),
        compiler_params=pltpu.CompilerParams(
            dimension_semantics=("arbitrary",)),
    )(ht, cb, fn, cn)


_NC = 2
_NS = 16
_NW = _NC * _NS
_ROWS_PER_W = N // _NW
_CHUNK = 72
_NCHUNK = _ROWS_PER_W // _CHUNK


def _gather_body(idx_hbm, cb_hbm, out_hbm, idx_v, rows_v, sem):
    wid = lax.axis_index("s") * _NC + lax.axis_index("c")
    base = wid * _ROWS_PER_W
    pltpu.sync_copy(idx_hbm.at[pl.ds(wid * _NCHUNK, _NCHUNK)], idx_v)
    for c in range(_NCHUNK):
        pltpu.async_copy(cb_hbm.at[idx_v.at[c]], rows_v, sem).wait()
        pltpu.sync_copy(rows_v, out_hbm.at[pl.ds(base + c * _CHUNK, _CHUNK)])


def _gather_rows(idx2d, cb):
    return pl.kernel(
        _gather_body,
        mesh=plsc.VectorSubcoreMesh(core_axis_name="c", subcore_axis_name="s"),
        out_type=jax.ShapeDtypeStruct((N, D), jnp.float32),
        scratch_types=[
            pltpu.VMEM((_NCHUNK, _CHUNK), jnp.int32),
            pltpu.VMEM((_CHUNK, D), jnp.float32),
            pltpu.SemaphoreType.DMA,
        ],
    )(idx2d, cb)


def kernel(h, cb):
    flat = jnp.transpose(h, (0, 2, 3, 1)).reshape(N, D)
    ht2 = -2.0 * jnp.transpose(h.reshape(B, D, H * W), (1, 0, 2)).reshape(D, N)
    fn = jnp.sum(flat * flat, axis=1).reshape(NT, 1, TT)
    cn = jnp.sum(cb * cb, axis=1).reshape(V, 1)
    idx = _nearest_codes(ht2, cb, fn, cn)
    idx_flat = idx.reshape(N)
    q = _gather_rows(idx_flat.reshape(_NW * _NCHUNK, _CHUNK), cb)
    z = idx_flat.reshape(B, H, W)
    return (z, q.reshape(B, H, W, D))


# speedup vs baseline: 2.9253x; 1.2524x over previous
import jax
import jax.numpy as jnp
from jax import lax
from jax.experimental import pallas as pl
from jax.experimental.pallas import tpu as pltpu
from jax.experimental.pallas import tpu_sc as plsc

B, D, H, W = 8, 256, 24, 24
N = B * H * W
V = 8192
KT = 512
NK = V // KT
TT = 512
NT = N // TT

_BIG = float(2**30)


_RB = 8


def _argmin_body(ht2_ref, cb_ref, fn_ref, cn_ref, out_ref):
    ht2 = ht2_ref[...]
    fn = fn_ref[0]
    acc_v = jnp.full((_RB, TT), jnp.inf, dtype=jnp.float32)
    acc_b = jnp.zeros((_RB, TT), dtype=jnp.float32)
    for k in range(NK):
        cbk = cb_ref[pl.ds(k * KT, KT), :]
        mm2 = lax.dot_general(cbk, ht2, (((1,), (0,)), ((), ())),
                              preferred_element_type=jnp.float32)
        cn = cn_ref[pl.ds(k * KT, KT), :]
        for r in range(KT // _RB):
            d2 = (fn + mm2[r * _RB:(r + 1) * _RB, :]) + cn[r * _RB:(r + 1) * _RB, :]
            upd = d2 < acc_v
            acc_v = jnp.where(upd, d2, acc_v)
            acc_b = jnp.where(upd, float(k * (KT // _RB) + r), acc_b)
    sub_iota = lax.broadcasted_iota(jnp.int32, (_RB, TT), 0).astype(jnp.float32)
    v, i = acc_v, acc_b * float(_RB) + sub_iota
    for s in (4, 2, 1):
        v1, v2 = v[:s], v[s:]
        i1, i2 = i[:s], i[s:]
        take2 = (v2 < v1) | ((v2 == v1) & (i2 < i1))
        v = jnp.where(take2, v2, v1)
        i = jnp.where(take2, i2, i1)
    out_ref[0] = i.astype(jnp.int32)


def _nearest_codes(ht, cb, fn, cn):
    return pl.pallas_call(
        _argmin_body,
        grid=(NT,),
        in_specs=[
            pl.BlockSpec((D, TT), lambda t: (0, t)),
            pl.BlockSpec((V, D), lambda t: (0, 0)),
            pl.BlockSpec((1, 1, TT), lambda t: (t, 0, 0)),
            pl.BlockSpec((V, 1), lambda t: (0, 0)),
        ],
        out_specs=pl.BlockSpec((1, 1, TT), lambda t: (t, 0, 0)),
        out_shape=jax.ShapeDtypeStruct((NT, 1, TT), jnp.int32),
        compiler_params=pltpu.CompilerParams(
            dimension_semantics=("arbitrary",)),
    )(ht, cb, fn, cn)


_NC = 2
_NS = 16
_NW = _NC * _NS
_ROWS_PER_W = N // _NW
_CHUNK = 72
_NCHUNK = _ROWS_PER_W // _CHUNK


def _gather_body(idx_hbm, cb_hbm, out_hbm, idx_v, rows_v, sem):
    wid = lax.axis_index("s") * _NC + lax.axis_index("c")
    base = wid * _ROWS_PER_W
    pltpu.sync_copy(idx_hbm.at[pl.ds(wid * _NCHUNK, _NCHUNK)], idx_v)
    for c in range(_NCHUNK):
        pltpu.async_copy(cb_hbm.at[idx_v.at[c]], rows_v, sem).wait()
        pltpu.sync_copy(rows_v, out_hbm.at[pl.ds(base + c * _CHUNK, _CHUNK)])


def _gather_rows(idx2d, cb):
    return pl.kernel(
        _gather_body,
        mesh=plsc.VectorSubcoreMesh(core_axis_name="c", subcore_axis_name="s"),
        out_type=jax.ShapeDtypeStruct((N, D), jnp.float32),
        scratch_types=[
            pltpu.VMEM((_NCHUNK, _CHUNK), jnp.int32),
            pltpu.VMEM((_CHUNK, D), jnp.float32),
            pltpu.SemaphoreType.DMA,
        ],
    )(idx2d, cb)


def kernel(h, cb):
    flat = jnp.transpose(h, (0, 2, 3, 1)).reshape(N, D)
    ht2 = -2.0 * jnp.transpose(h.reshape(B, D, H * W), (1, 0, 2)).reshape(D, N)
    fn = jnp.sum(flat * flat, axis=1).reshape(NT, 1, TT)
    cn = jnp.sum(cb * cb, axis=1).reshape(V, 1)
    idx = _nearest_codes(ht2, cb, fn, cn)
    idx_flat = idx.reshape(N)
    q = _gather_rows(idx_flat.reshape(_NW * _NCHUNK, _CHUNK), cb)
    z = idx_flat.reshape(B, H, W)
    return (z, q.reshape(B, H, W, D))
